# baseline (speedup 1.0000x reference)
import functools

import jax
import jax.numpy as jnp
from jax import lax
from jax.experimental import pallas as pl
from jax.experimental.pallas import tpu as pltpu
from jax.experimental.pallas import tpu_sc as plsc

D_SIZES = (10, 10, 20, 5, 10)
W_SIZES = (16, 16, 32, 16, 16)
_COL_BASE = (0, 10, 20, 40, 45)
NUM_CODES = 100000
BATCH = 16384
LANE = 128

_info = plsc.get_sparse_core_info()
_NC = _info.num_cores
_NS = _info.num_subcores
_NW = _NC * _NS
_BPW = BATCH // _NW
_CH = 128
_NCH = _BPW // _CH
_RPT = 6280
_RSTRIDE = 6248
_WCH = 512
_CHUNKS = tuple((k * _WCH, _WCH) for k in range(12)) + ((12 * _WCH, 136),)

_mesh = plsc.VectorSubcoreMesh(core_axis_name="c", subcore_axis_name="s")


@functools.partial(
    pl.kernel,
    mesh=_mesh,
    out_type=(
        jax.ShapeDtypeStruct((BATCH, LANE), jnp.float32),
        *[jax.ShapeDtypeStruct((_NC * NUM_CODES, wd), jnp.float32)
          for wd in W_SIZES],
    ),
    scratch_types=[
        pltpu.VMEM((_NCH, _CH), jnp.int32),
        pltpu.VMEM((_NCH, _CH), jnp.int32),
        *[pltpu.VMEM((_WCH, d), jnp.float32) for d in D_SIZES],
        *[pltpu.VMEM((_WCH, wd), jnp.float32) for wd in W_SIZES],
        *[pltpu.VMEM((_CH, wd), jnp.float32) for wd in W_SIZES],
        pltpu.VMEM((_CH, LANE), jnp.float32),
        pltpu.SemaphoreType.DMA,
    ],
    compiler_params=pltpu.CompilerParams(use_tc_tiling_on_sc=False),
)
def _gather5(ids_hbm, w0, w1, w2, w3, w4,
             out, r0, r1, r2, r3, r4,
             idx_v, sidx_v, n0, n1, n2, n3, n4, v0, v1, v2, v3, v4,
             g0, g1, g2, g3, g4, stage, sem):
    cid = lax.axis_index("c")
    sid = lax.axis_index("s")
    wid = sid * _NC + cid
    tabs = (w0, w1, w2, w3, w4)
    wides = (r0, r1, r2, r3, r4)
    nars = (n0, n1, n2, n3, n4)
    wbufs = (v0, v1, v2, v3, v4)
    gbufs = (g0, g1, g2, g3, g4)

    rows0 = sid * _RSTRIDE
    for h0, hn in _CHUNKS:
        src0 = rows0 + h0
        for t, (w, d) in enumerate(zip(tabs, D_SIZES)):
            pltpu.sync_copy(w.at[pl.ds(src0, hn)], nars[t].at[pl.ds(0, hn)])

        def widen_row(r):
            z = r - r
            for t, (d, wd) in enumerate(zip(D_SIZES, W_SIZES)):
                wbufs[t][r, pl.ds(0, 16)] = nars[t][r, pl.ds(z, 16)]
                if wd > 16:
                    wbufs[t][r, pl.ds(16, 16)] = nars[t][r, pl.ds(z + 16, 16)]
        pl.loop(0, hn)(widen_row)

        for t, wr in enumerate(wides):
            pltpu.sync_copy(wbufs[t].at[pl.ds(0, hn)],
                            wr.at[pl.ds(cid * NUM_CODES + src0, hn)])
    plsc.subcore_barrier()

    pltpu.sync_copy(ids_hbm.at[pl.ds(wid * _NCH, _NCH)], idx_v)
    coff = cid * NUM_CODES
    for j in range(_NCH):
        for g in range(_CH // 16):
            sidx_v[j, pl.ds(g * 16, 16)] = idx_v[j, pl.ds(g * 16, 16)] + coff
    base = wid * _BPW

    for j in range(_NCH):
        copies = [
            pltpu.async_copy(wr.at[sidx_v.at[j]], gbufs[t], sem)
            for t, wr in enumerate(wides)
        ]
        for c in copies:
            c.wait()

        def group_body(g, _j=j):
            for l in range(16):
                i = g * 16 + l
                stage[i, pl.ds(0, 16)] = g0[i, pl.ds(0, 16)]
                stage[i, pl.ds(10, 16)] = g1[i, pl.ds(0, 16)]
                stage[i, pl.ds(20, 16)] = g2[i, pl.ds(0, 16)]
                stage[i, pl.ds(36, 16)] = g2[i, pl.ds(16, 16)]
                stage[i, pl.ds(40, 16)] = g3[i, pl.ds(0, 16)]
                stage[i, pl.ds(45, 16)] = g4[i, pl.ds(0, 16)]
        pl.loop(0, _CH // 16)(group_body)
        pltpu.sync_copy(stage, out.at[pl.ds(base + j * _CH, _CH)])


def kernel(code_ids, W0, W1, W2, W3, W4):
    ids2d = code_ids.astype(jnp.int32).reshape(BATCH // _CH, _CH)
    packed = _gather5(ids2d, W0, W1, W2, W3, W4)[0]
    return tuple(
        packed[:, cb:cb + d] for cb, d in zip(_COL_BASE, D_SIZES)
    )


# speedup vs baseline: 7.0325x; 7.0325x over previous
import functools

import jax
import jax.numpy as jnp
from jax import lax
from jax.experimental import pallas as pl
from jax.experimental.pallas import tpu as pltpu
from jax.experimental.pallas import tpu_sc as plsc

D_SIZES = (10, 10, 20, 5, 10)
_COL_BASE = (0, 10, 20, 40, 45)
D_SUM = 55
NUM_CODES = 100000
BATCH = 16384

_info = plsc.get_sparse_core_info()
_NC = _info.num_cores
_NS = _info.num_subcores
_NW = _NC * _NS
_BPW = BATCH // _NW
_CH = 128
_NCH = _BPW // _CH

_mesh = plsc.VectorSubcoreMesh(core_axis_name="c", subcore_axis_name="s")


@functools.partial(
    pl.kernel,
    mesh=_mesh,
    out_type=jax.ShapeDtypeStruct((D_SUM, BATCH), jnp.float32),
    scratch_types=[
        pltpu.VMEM((_NCH, _CH), jnp.int32),
        pltpu.VMEM((D_SUM, _CH), jnp.float32),
        pltpu.SemaphoreType.DMA,
    ],
    compiler_params=pltpu.CompilerParams(use_tc_tiling_on_sc=False),
)
def _gather5(ids_hbm, w0t, w1t, w2t, w3t, w4t, out, idx_v, blk, sem):
    wid = lax.axis_index("s") * _NC + lax.axis_index("c")
    tabs = (w0t, w1t, w2t, w3t, w4t)

    pltpu.sync_copy(ids_hbm.at[pl.ds(wid * _NCH, _NCH)], idx_v)
    base = wid * _BPW

    for j in range(_NCH):
        idx_chunk = idx_v.at[j]
        copies = []
        k = 0
        for t, d in enumerate(D_SIZES):
            for f in range(d):
                copies.append(
                    pltpu.async_copy(tabs[t].at[f].at[idx_chunk],
                                     blk.at[k], sem))
                k += 1
        for c in copies:
            c.wait()
        pltpu.sync_copy(blk, out.at[:, pl.ds(base + j * _CH, _CH)])


def kernel(code_ids, W0, W1, W2, W3, W4):
    ids2d = code_ids.astype(jnp.int32).reshape(BATCH // _CH, _CH)
    packed = _gather5(ids2d, W0.T, W1.T, W2.T, W3.T, W4.T)
    return tuple(
        packed[cb:cb + d, :].T for cb, d in zip(_COL_BASE, D_SIZES)
    )


# speedup vs baseline: 7.3531x; 1.0456x over previous
import functools

import jax
import jax.numpy as jnp
from jax import lax
from jax.experimental import pallas as pl
from jax.experimental.pallas import tpu as pltpu
from jax.experimental.pallas import tpu_sc as plsc

D_SIZES = (10, 10, 20, 5, 10)
_COL_BASE = (0, 10, 20, 40, 45)
D_SUM = 55
NUM_CODES = 100000
BATCH = 16384

_info = plsc.get_sparse_core_info()
_NC = _info.num_cores
_NS = _info.num_subcores
_NW = _NC * _NS
_BPW = BATCH // _NW
_CH = 128
_NCH = _BPW // _CH

_mesh = plsc.VectorSubcoreMesh(core_axis_name="c", subcore_axis_name="s")


@functools.partial(
    pl.kernel,
    mesh=_mesh,
    out_type=jax.ShapeDtypeStruct((D_SUM, BATCH), jnp.float32),
    scratch_types=[
        pltpu.VMEM((_NCH, _CH), jnp.int32),
        pltpu.VMEM((D_SUM, _BPW), jnp.float32),
        pltpu.SemaphoreType.DMA,
    ],
    compiler_params=pltpu.CompilerParams(use_tc_tiling_on_sc=False),
)
def _gather5(ids_hbm, w0t, w1t, w2t, w3t, w4t, out, idx_v, blk, sem):
    wid = lax.axis_index("s") * _NC + lax.axis_index("c")
    tabs = (w0t, w1t, w2t, w3t, w4t)

    pltpu.sync_copy(ids_hbm.at[pl.ds(wid * _NCH, _NCH)], idx_v)
    base = wid * _BPW

    copies = []
    for j in range(_NCH):
        idx_chunk = idx_v.at[j]
        k = 0
        for t, d in enumerate(D_SIZES):
            for f in range(d):
                copies.append(
                    pltpu.async_copy(
                        tabs[t].at[f].at[idx_chunk],
                        blk.at[k, pl.ds(j * _CH, _CH)], sem))
                k += 1
    for c in copies:
        c.wait()
    pltpu.sync_copy(blk, out.at[:, pl.ds(base, _BPW)])


def kernel(code_ids, W0, W1, W2, W3, W4):
    ids2d = code_ids.astype(jnp.int32).reshape(BATCH // _CH, _CH)
    packed = _gather5(ids2d, W0.T, W1.T, W2.T, W3.T, W4.T)
    return tuple(
        packed[cb:cb + d, :].T for cb, d in zip(_COL_BASE, D_SIZES)
    )
